# Initial kernel scaffold; baseline (speedup 1.0000x reference)
import jax
import jax.numpy as jnp
from jax.experimental import pallas as pl


def kernel(input, table):
    raise NotImplementedError("write your pallas kernel here")


# speedup vs baseline: 1.0409x; 1.0409x over previous
import functools

import jax
import jax.numpy as jnp
from jax import lax
from jax.experimental import pallas as pl
from jax.experimental.pallas import tpu as pltpu
from jax.experimental.pallas import tpu_sc as plsc

_VOCAB = 1000000
_EMBED = 64
_NC = 2
_NS = 16
_NW = _NC * _NS
_CH = 128


def _make_emb_kernel(n_ch: int, b_per_w: int):
    mesh = plsc.VectorSubcoreMesh(core_axis_name="c", subcore_axis_name="s")

    @functools.partial(
        pl.kernel,
        mesh=mesh,
        compiler_params=pltpu.CompilerParams(use_tc_tiling_on_sc=False),
        out_type=jax.ShapeDtypeStruct((_NW * b_per_w, _EMBED), jnp.float32),
        scratch_types=[
            pltpu.VMEM((n_ch, _CH), jnp.int32),
            pltpu.VMEM((_CH, _EMBED), jnp.float32),
            pltpu.VMEM((_CH, _EMBED), jnp.float32),
            pltpu.SemaphoreType.DMA,
            pltpu.SemaphoreType.DMA,
        ],
    )
    def emb(idx_hbm, table_hbm, out_hbm, idx_v, rows0, rows1, sem0, sem1):
        wid = lax.axis_index("s") * _NC + lax.axis_index("c")
        base = wid * b_per_w
        pltpu.sync_copy(idx_hbm.at[wid], idx_v)

        n_pair = n_ch // 2

        def gather(j, rows, sem):
            pltpu.async_copy(table_hbm.at[idx_v.at[j]], rows, sem)

        def store(j, rows, sem):
            pltpu.make_async_copy(table_hbm.at[idx_v.at[j]], rows, sem).wait()
            pltpu.sync_copy(rows, out_hbm.at[pl.ds(base + j * _CH, _CH)])

        gather(0, rows0, sem0)

        def body(i, _):
            j0 = 2 * i
            gather(j0 + 1, rows1, sem1)
            store(j0, rows0, sem0)

            @pl.when(i + 1 < n_pair)
            def _():
                gather(j0 + 2, rows0, sem0)

            store(j0 + 1, rows1, sem1)
            return 0

        lax.fori_loop(0, n_pair, body, 0)

    return emb


def kernel(input, table):
    bsz, seq = input.shape
    total = bsz * seq
    b_per_w = total // _NW
    n_ch = b_per_w // _CH
    idx = input.reshape(_NW, n_ch, _CH).astype(jnp.int32)
    out = _make_emb_kernel(n_ch, b_per_w)(idx, table)
    return out.reshape(bsz, seq, _EMBED)


# speedup vs baseline: 1.0626x; 1.0209x over previous
import functools

import jax
import jax.numpy as jnp
from jax import lax
from jax.experimental import pallas as pl
from jax.experimental.pallas import tpu as pltpu
from jax.experimental.pallas import tpu_sc as plsc

_VOCAB = 1000000
_EMBED = 64
_NC = 2
_NS = 16
_NW = _NC * _NS
_CH = 128
_NBUF = 8


def _make_emb_kernel(n_ch: int, b_per_w: int):
    mesh = plsc.VectorSubcoreMesh(core_axis_name="c", subcore_axis_name="s")

    @functools.partial(
        pl.kernel,
        mesh=mesh,
        compiler_params=pltpu.CompilerParams(use_tc_tiling_on_sc=False),
        out_type=jax.ShapeDtypeStruct((_NW * b_per_w, _EMBED), jnp.float32),
        scratch_types=[
            pltpu.VMEM((n_ch, _CH), jnp.int32),
            pltpu.VMEM((_NBUF, _CH, _EMBED), jnp.float32),
            pltpu.SemaphoreType.DMA((_NBUF,)),
            pltpu.SemaphoreType.DMA((_NBUF,)),
        ],
    )
    def emb(idx_hbm, table_hbm, out_hbm, idx_v, rows, gsem, ssem):
        wid = lax.axis_index("s") * _NC + lax.axis_index("c")
        base = wid * b_per_w
        pltpu.sync_copy(idx_hbm.at[wid], idx_v)

        def gather(j, b):
            pltpu.async_copy(table_hbm.at[idx_v.at[j]], rows.at[b], gsem.at[b])

        def gwait(j, b):
            pltpu.make_async_copy(
                table_hbm.at[idx_v.at[j]], rows.at[b], gsem.at[b]
            ).wait()

        def store(j, b):
            pltpu.async_copy(
                rows.at[b], out_hbm.at[pl.ds(base + j * _CH, _CH)], ssem.at[b]
            )

        def swait(j, b):
            pltpu.make_async_copy(
                rows.at[b], out_hbm.at[pl.ds(base + j * _CH, _CH)], ssem.at[b]
            ).wait()

        for b in range(_NBUF):
            gather(b, b)

        def body(i, _):
            j0 = i * _NBUF
            for b in range(_NBUF):
                j = j0 + b
                gwait(j, b)
                store(j, b)
                swait(j, b)

                @pl.when(j + _NBUF < n_ch)
                def _():
                    gather(j + _NBUF, b)

            return 0

        lax.fori_loop(0, n_ch // _NBUF, body, 0)

    return emb


def kernel(input, table):
    bsz, seq = input.shape
    total = bsz * seq
    b_per_w = total // _NW
    n_ch = b_per_w // _CH
    idx = input.reshape(_NW, n_ch, _CH).astype(jnp.int32)
    out = _make_emb_kernel(n_ch, b_per_w)(idx, table)
    return out.reshape(bsz, seq, _EMBED)
